# baseline (speedup 1.0000x reference)
import functools

import jax
import jax.numpy as jnp
from jax import lax
from jax.experimental import pallas as pl
from jax.experimental.pallas import tpu as pltpu
from jax.experimental.pallas import tpu_sc as plsc

NC = 2
NS = 16
NW = NC * NS
L = 16
SS = 100
ZR = 800
W = 8


def _rsqrt(q):
    i = lax.bitcast_convert_type(q, jnp.int32)
    i = 0x5F3759DF - lax.shift_right_arithmetic(i, 1)
    y = lax.bitcast_convert_type(i, jnp.float32)
    for _ in range(3):
        y = y * (1.5 - 0.5 * q * y * y)
    return y


def _make_edge_kernel(E, NPAD, C):
    EPW = E // NW
    NCH = EPW // C
    G = C // L
    K100 = C // SS
    RPT = NPAD // NS
    KR = RPT // SS
    ZK = ZR // SS

    mesh = plsc.VectorSubcoreMesh(core_axis_name="c", subcore_axis_name="s",
                                  num_cores=NC, num_subcores=NS)
    ef32 = jax.ShapeDtypeStruct((E,), jnp.float32)

    @functools.partial(
        pl.kernel,
        out_type=[ef32] * 7 + [jax.ShapeDtypeStruct((NC, NPAD, W), jnp.float32)],
        mesh=mesh,
        compiler_params=pltpu.CompilerParams(
            needs_layout_passes=False, use_tc_tiling_on_sc=False),
        scratch_types=[
            pltpu.VMEM((2 * (C // SS), SS), jnp.int32),
            pltpu.VMEM((2 * (C // SS), SS), jnp.int32),
            pltpu.VMEM((2 * C, 8), jnp.float32),
            pltpu.VMEM((2 * C, 8), jnp.float32),
            pltpu.VMEM((2 * C,), jnp.float32),
            pltpu.VMEM((2 * C,), jnp.float32),
            pltpu.VMEM((2 * C,), jnp.float32),
            pltpu.VMEM((2 * C,), jnp.float32),
            pltpu.VMEM((2 * C,), jnp.float32),
            pltpu.VMEM((2 * C,), jnp.float32),
            pltpu.VMEM((2 * C,), jnp.float32),
            pltpu.VMEM((2 * C,), jnp.float32),
            pltpu.VMEM((2 * C,), jnp.float32),
            pltpu.VMEM((2 * C,), jnp.float32),
            pltpu.VMEM((2 * C, W), jnp.float32),
            pltpu.VMEM((2 * C, W), jnp.float32),
            pltpu.VMEM((96,), jnp.float32),
            pltpu.VMEM((ZR, W), jnp.float32),
            pltpu.VMEM((NPAD // NS // SS, SS), jnp.int32),
            pltpu.VMEM_SHARED((NPAD, W), jnp.float32),
            pltpu.SemaphoreType.DMA,
            pltpu.SemaphoreType.DMA,
            pltpu.SemaphoreType.DMA,
            pltpu.SemaphoreType.DMA,
            pltpu.SemaphoreType.DMA,
        ],
    )
    def edge_kernel(tab, na2d, nb2d, pE, pA, pI, consts, zeros8, ids2d,
                    l0_o, cc_o, ss_o, Ne_o, M1_o, M2_o, Ve_o, accp,
                    naidx, nbidx, rowsA, rowsB, pEv, pAv, pIv,
                    l0v, ccv, ssv, Nev, M1v, M2v, Vev, updA, updB, cvec,
                    zbuf, idbuf, acc_sh, semIn, semA, semB, semOut, semU):
        cid = lax.axis_index("c")
        sid = lax.axis_index("s")
        wid = sid * NC + cid

        pltpu.sync_copy(zeros8, zbuf)
        pltpu.sync_copy(ids2d.at[pl.ds(sid * KR, KR)], idbuf)

        def zfire(j, carry):
            pltpu.async_copy(
                zbuf.at[pl.ds(0, SS)], acc_sh.at[idbuf.at[j]], semU)
            return carry
        lax.fori_loop(0, KR, zfire, 0)

        def zdrain(j, carry):
            pltpu.make_async_copy(
                zbuf.at[pl.ds(0, SS)], acc_sh.at[idbuf.at[0]], semU).wait()
            return carry
        lax.fori_loop(0, KR, zdrain, 0)
        pltpu.sync_copy(consts, cvec)

        iot = lax.iota(jnp.int32, L)
        zf = jnp.zeros((L,), jnp.float32)

        def zpad(g, carry):
            r = g * L + iot
            for col in range(3, W):
                cvec_i = jnp.full((L,), col, jnp.int32)
                plsc.store_scatter(updA, [r, cvec_i], zf)
                plsc.store_scatter(updB, [r, cvec_i], zf)
            return carry
        lax.fori_loop(0, 2 * G, zpad, 0)

        plsc.subcore_barrier()

        s_ax = cvec[pl.ds(0, L)]
        s_bend = cvec[pl.ds(16, L)]
        s_sw = cvec[pl.ds(32, L)]
        s_mw = cvec[pl.ds(48, L)]
        Fc = cvec[pl.ds(64, L)]
        Mc = cvec[pl.ds(80, L)]

        base = wid * EPW
        base100 = wid * (EPW // SS)

        def fire_inputs(k):
            par = lax.rem(k, 2)
            off = base + k * C
            off100 = base100 + k * K100
            pltpu.async_copy(na2d.at[pl.ds(off100, K100)],
                             naidx.at[pl.ds(par * K100, K100)], semIn)
            pltpu.async_copy(nb2d.at[pl.ds(off100, K100)],
                             nbidx.at[pl.ds(par * K100, K100)], semIn)
            pltpu.async_copy(pE.at[pl.ds(off, C)],
                             pEv.at[pl.ds(par * C, C)], semIn)
            pltpu.async_copy(pA.at[pl.ds(off, C)],
                             pAv.at[pl.ds(par * C, C)], semIn)
            pltpu.async_copy(pI.at[pl.ds(off, C)],
                             pIv.at[pl.ds(par * C, C)], semIn)

        def drain_inputs():
            pltpu.make_async_copy(
                na2d.at[pl.ds(0, K100)], naidx.at[pl.ds(0, K100)],
                semIn).wait()
            pltpu.make_async_copy(
                nb2d.at[pl.ds(0, K100)], nbidx.at[pl.ds(0, K100)],
                semIn).wait()
            pltpu.make_async_copy(
                pE.at[pl.ds(0, C)], pEv.at[pl.ds(0, C)], semIn).wait()
            pltpu.make_async_copy(
                pA.at[pl.ds(0, C)], pAv.at[pl.ds(0, C)], semIn).wait()
            pltpu.make_async_copy(
                pI.at[pl.ds(0, C)], pIv.at[pl.ds(0, C)], semIn).wait()

        def fire_gathers(k):
            par = lax.rem(k, 2)

            def fire(j, carry2):
                pltpu.async_copy(
                    tab.at[naidx.at[par * K100 + j]],
                    rowsA.at[pl.ds(par * C + j * SS, SS)], semA)
                pltpu.async_copy(
                    tab.at[nbidx.at[par * K100 + j]],
                    rowsB.at[pl.ds(par * C + j * SS, SS)], semB)
                return carry2
            lax.fori_loop(0, K100, fire, 0)

        def drain_gathers():
            def drain(j, carry2):
                pltpu.make_async_copy(
                    tab.at[naidx.at[0]], rowsA.at[pl.ds(0, SS)], semA).wait()
                pltpu.make_async_copy(
                    tab.at[nbidx.at[0]], rowsB.at[pl.ds(0, SS)], semB).wait()
                return carry2
            lax.fori_loop(0, K100, drain, 0)

        def drain_outs():
            for o in (l0_o, cc_o, ss_o, Ne_o, M1_o, M2_o, Ve_o):
                pltpu.make_async_copy(
                    l0v.at[pl.ds(0, C)], o.at[pl.ds(base, C)], semOut).wait()

        def drain_scatter():
            def udrain(j, carry2):
                pltpu.make_async_copy(
                    updA.at[pl.ds(0, SS)], acc_sh.at[naidx.at[0]],
                    semU).wait()
                pltpu.make_async_copy(
                    updB.at[pl.ds(0, SS)], acc_sh.at[nbidx.at[0]],
                    semU).wait()
                return carry2
            lax.fori_loop(0, K100, udrain, 0)

        fire_inputs(0)
        drain_inputs()
        fire_gathers(0)

        def chunk(k, carry):
            par = lax.rem(k, 2)
            off = base + k * C

            @pl.when(k > 0)
            def _():
                drain_outs()
                drain_scatter()

            @pl.when(k + 1 < NCH)
            def _():
                fire_inputs(k + 1)

            drain_gathers()

            def grp(g, carry2):
                r = par * C + g * L + iot

                def gA(col):
                    return plsc.load_gather(
                        rowsA, [r, jnp.full((L,), col, jnp.int32)])

                def gB(col):
                    return plsc.load_gather(
                        rowsB, [r, jnp.full((L,), col, jnp.int32)])

                d0 = gA(0); d1 = gA(1); ta = -gA(2)
                d3 = gB(0); d4 = gB(1); tb = -gB(2)
                cxA = gA(3); czA = gA(4)
                cxB = gB(3); czB = gB(4)
                dx = cxB - cxA
                dz = czB - czA
                q = dx * dx + dz * dz
                y = _rsqrt(q)
                y2 = y * y
                y3 = y2 * y
                sl = pl.ds(par * C + g * L, L)
                eE = pEv[sl]; eA = pAv[sl]; eI = pIv[sl]
                EAv = eE * eA
                EIv = eE * eI
                k_ax = EAv * s_ax * y
                k_bend = EIv * s_bend * y
                k_sw = EIv * s_sw * y2
                k_tr = EIv * s_ax * y3
                k_mw = EIv * s_mw * y2
                cc = dx * y
                ss = dz * y
                ua = cc * d0 + ss * d1
                wa = cc * d1 - ss * d0
                ub = cc * d3 + ss * d4
                wb = cc * d4 - ss * d3
                f0 = k_ax * (ua - ub)
                wab = wa - wb
                tsum = ta + tb
                f1 = 12.0 * k_tr * wab + 6.0 * k_sw * tsum
                mwab = 6.0 * k_mw * wab
                f2 = mwab + k_bend * (4.0 * ta + 2.0 * tb)
                f5 = mwab + k_bend * (2.0 * ta + 4.0 * tb)
                gx = cc * f0 - ss * f1
                gy = ss * f0 + cc * f1
                l0v[sl] = q * y
                ccv[sl] = cc
                ssv[sl] = ss
                Nev[sl] = -f0 * Fc
                M1v[sl] = f2 * Mc
                M2v[sl] = f5 * Mc
                Vev[sl] = -f1 * Fc

                def cS(i):
                    return jnp.full((L,), i, jnp.int32)
                plsc.store_scatter(updA, [r, cS(0)], gx)
                plsc.store_scatter(updA, [r, cS(1)], gy)
                plsc.store_scatter(updA, [r, cS(2)], f2)
                plsc.store_scatter(updB, [r, cS(0)], -gx)
                plsc.store_scatter(updB, [r, cS(1)], -gy)
                plsc.store_scatter(updB, [r, cS(2)], f5)
                return carry2

            lax.fori_loop(0, G, grp, 0)

            ds_o = pl.ds(off, C)
            ds_p = pl.ds(par * C, C)
            pltpu.async_copy(l0v.at[ds_p], l0_o.at[ds_o], semOut)
            pltpu.async_copy(ccv.at[ds_p], cc_o.at[ds_o], semOut)
            pltpu.async_copy(ssv.at[ds_p], ss_o.at[ds_o], semOut)
            pltpu.async_copy(Nev.at[ds_p], Ne_o.at[ds_o], semOut)
            pltpu.async_copy(M1v.at[ds_p], M1_o.at[ds_o], semOut)
            pltpu.async_copy(M2v.at[ds_p], M2_o.at[ds_o], semOut)
            pltpu.async_copy(Vev.at[ds_p], Ve_o.at[ds_o], semOut)

            def ufire(j, carry2):
                pltpu.async_copy(
                    updA.at[pl.ds(par * C + j * SS, SS)],
                    acc_sh.at[naidx.at[par * K100 + j]], semU, add=True)
                pltpu.async_copy(
                    updB.at[pl.ds(par * C + j * SS, SS)],
                    acc_sh.at[nbidx.at[par * K100 + j]], semU, add=True)
                return carry2
            lax.fori_loop(0, K100, ufire, 0)

            @pl.when(k + 1 < NCH)
            def _():
                drain_inputs()
                fire_gathers(k + 1)
            return carry

        lax.fori_loop(0, NCH, chunk, 0)
        drain_outs()
        drain_scatter()

        plsc.subcore_barrier()
        for r in range(RPT // ZR):
            def gfire(j, carry, r=r):
                pltpu.async_copy(
                    acc_sh.at[idbuf.at[r * ZK + j]],
                    zbuf.at[pl.ds(j * SS, SS)], semU)
                return carry
            lax.fori_loop(0, ZK, gfire, 0)

            def gdrain(j, carry):
                pltpu.make_async_copy(
                    acc_sh.at[idbuf.at[0]], zbuf.at[pl.ds(0, SS)],
                    semU).wait()
                return carry
            lax.fori_loop(0, ZK, gdrain, 0)
            pltpu.sync_copy(
                zbuf, accp.at[cid, pl.ds(sid * RPT + r * ZR, ZR)])

    return edge_kernel


def _make_node_kernel(NPAD, CH2):
    FPT4 = NPAD * 4 // NW
    CH8 = CH2 * 2
    NCH2 = FPT4 // CH2
    G2 = CH2 // L
    G8 = CH8 // L

    mesh = plsc.VectorSubcoreMesh(core_axis_name="c", subcore_axis_name="s",
                                  num_cores=NC, num_subcores=NS)

    @functools.partial(
        pl.kernel,
        out_type=[jax.ShapeDtypeStruct((NPAD * 8,), jnp.float32),
                  jax.ShapeDtypeStruct((NPAD * 4,), jnp.float32),
                  jax.ShapeDtypeStruct((NPAD * 4,), jnp.float32)],
        mesh=mesh,
        compiler_params=pltpu.CompilerParams(
            needs_layout_passes=False, use_tc_tiling_on_sc=False),
        scratch_types=[
            pltpu.VMEM((CH8,), jnp.float32),
            pltpu.VMEM((CH8,), jnp.float32),
            pltpu.VMEM((CH8,), jnp.float32),
            pltpu.VMEM((CH2,), jnp.float32),
            pltpu.VMEM((CH2,), jnp.float32),
            pltpu.VMEM((CH2,), jnp.float32),
            pltpu.VMEM((CH2,), jnp.float32),
            pltpu.VMEM((32,), jnp.float32),
        ],
    )
    def node_kernel(accpf, pr4f, fx4f, scl, nf_o, pd_o, fe_o,
                    a0v, a1v, nfv, prv, fxv, pdv, fev, sclv):
        cid = lax.axis_index("c")
        sid = lax.axis_index("s")
        wid = sid * NC + cid
        pltpu.sync_copy(scl, sclv)
        sdisp = sclv[pl.ds(0, L)]
        sfext = sclv[pl.ds(16, L)]
        base4 = wid * FPT4
        base8 = wid * FPT4 * 2

        def chunk(k, carry):
            off4 = base4 + k * CH2
            off8 = base8 + k * CH8
            pltpu.sync_copy(accpf.at[pl.ds(off8, CH8)], a0v)
            pltpu.sync_copy(accpf.at[pl.ds(NPAD * 8 + off8, CH8)], a1v)
            pltpu.sync_copy(pr4f.at[pl.ds(off4, CH2)], prv)
            pltpu.sync_copy(fx4f.at[pl.ds(off4, CH2)], fxv)

            def grp8(g, carry2):
                slg = pl.ds(g * L, L)
                nfv[slg] = a0v[slg] + a1v[slg]
                return carry2
            lax.fori_loop(0, G8, grp8, 0)

            def grp4(g, carry2):
                slg = pl.ds(g * L, L)
                pdv[slg] = prv[slg] * sdisp
                fev[slg] = fxv[slg] * sfext
                return carry2
            lax.fori_loop(0, G2, grp4, 0)

            pltpu.sync_copy(nfv, nf_o.at[pl.ds(off8, CH8)])
            pltpu.sync_copy(pdv, pd_o.at[pl.ds(off4, CH2)])
            pltpu.sync_copy(fev, fe_o.at[pl.ds(off4, CH2)])
            return carry

        lax.fori_loop(0, NCH2, chunk, 0)

    return node_kernel


def kernel(pred_raw, connectivity, coords, prop_E, prop_A, prop_I22,
           F_ext, u_c, theta_c, F_c, M_c):
    E = connectivity.shape[0]
    Nn = pred_raw.shape[0]
    C = 400
    NPAD = 102400

    nA = connectivity[:, 0].reshape(E // SS, SS)
    nB = connectivity[:, 1].reshape(E // SS, SS)
    tab = jnp.concatenate(
        [pred_raw, coords[:, 0:1], coords[:, 2:3],
         jnp.zeros((Nn, 3), jnp.float32)], axis=1)
    zeros8 = jnp.zeros((ZR, W), jnp.float32)

    def sp(v):
        return jnp.full((L,), v, jnp.float32)
    consts = jnp.concatenate([
        sp(u_c / F_c), sp(theta_c / M_c), sp(theta_c / F_c),
        sp(u_c / M_c), sp(F_c), sp(M_c)])

    z = jnp.float32(0.0)
    scl = jnp.concatenate([
        jnp.tile(jnp.stack([u_c, u_c, theta_c, z]), 4),
        jnp.tile(jnp.stack([1.0 / F_c, 1.0 / F_c, 1.0 / M_c, z]), 4)])

    pr4 = jnp.pad(pred_raw, ((0, NPAD - Nn), (0, 1)))
    fx4 = jnp.pad(F_ext, ((0, NPAD - Nn), (0, 1)))
    ids2d = jnp.arange(NPAD, dtype=jnp.int32).reshape(NPAD // SS, SS)

    edge_k = _make_edge_kernel(E, NPAD, C)
    l0, cc, ss, Ne, M1, M2, Ve, accp = edge_k(
        tab, nA, nB, prop_E, prop_A, prop_I22, consts, zeros8, ids2d)

    node_k = _make_node_kernel(NPAD, 3200)
    nf8f, pd4f, fe4f = node_k(
        accp.reshape(-1), pr4.reshape(-1), fx4.reshape(-1), scl)

    nf = nf8f.reshape(NPAD, W)[:Nn, :3]
    pd = pd4f.reshape(NPAD, 4)[:Nn, :3]
    fe = fe4f.reshape(NPAD, 4)[:Nn, :3]
    return (nf, fe, Ne, M1, M2, Ve, pd, l0, cc, ss)


# speedup vs baseline: 1.1860x; 1.1860x over previous
import functools

import jax
import jax.numpy as jnp
from jax import lax
from jax.experimental import pallas as pl
from jax.experimental.pallas import tpu as pltpu
from jax.experimental.pallas import tpu_sc as plsc

NC = 2
NS = 16
NW = NC * NS
L = 16
SS = 100
ZR = 800
W = 8


def _rsqrt(q):
    i = lax.bitcast_convert_type(q, jnp.int32)
    i = 0x5F3759DF - lax.shift_right_arithmetic(i, 1)
    y = lax.bitcast_convert_type(i, jnp.float32)
    for _ in range(3):
        y = y * (1.5 - 0.5 * q * y * y)
    return y


def _make_edge_kernel(E, NPAD, C):
    EPW = E // NW
    NCH = EPW // C
    G = C // L
    K100 = C // SS
    RPT = NPAD // NS
    KR = RPT // SS
    ZK = ZR // SS

    mesh = plsc.VectorSubcoreMesh(core_axis_name="c", subcore_axis_name="s",
                                  num_cores=NC, num_subcores=NS)
    ef32 = jax.ShapeDtypeStruct((E,), jnp.float32)

    @functools.partial(
        pl.kernel,
        out_type=[ef32] * 7 + [jax.ShapeDtypeStruct((NC, NPAD, W), jnp.float32)],
        mesh=mesh,
        compiler_params=pltpu.CompilerParams(
            needs_layout_passes=False, use_tc_tiling_on_sc=False),
        scratch_types=[
            pltpu.VMEM((3 * (C // SS), SS), jnp.int32),
            pltpu.VMEM((3 * (C // SS), SS), jnp.int32),
            pltpu.VMEM((3 * C, 8), jnp.float32),
            pltpu.VMEM((3 * C, 8), jnp.float32),
            pltpu.VMEM((3 * C,), jnp.float32),
            pltpu.VMEM((3 * C,), jnp.float32),
            pltpu.VMEM((3 * C,), jnp.float32),
            pltpu.VMEM((2 * C,), jnp.float32),
            pltpu.VMEM((2 * C,), jnp.float32),
            pltpu.VMEM((2 * C,), jnp.float32),
            pltpu.VMEM((2 * C,), jnp.float32),
            pltpu.VMEM((2 * C,), jnp.float32),
            pltpu.VMEM((2 * C,), jnp.float32),
            pltpu.VMEM((2 * C,), jnp.float32),
            pltpu.VMEM((2 * C, W), jnp.float32),
            pltpu.VMEM((2 * C, W), jnp.float32),
            pltpu.VMEM((96,), jnp.float32),
            pltpu.VMEM((ZR, W), jnp.float32),
            pltpu.VMEM((NPAD // NS // SS, SS), jnp.int32),
            pltpu.VMEM_SHARED((NPAD, W), jnp.float32),
            pltpu.SemaphoreType.DMA,
            pltpu.SemaphoreType.DMA,
            pltpu.SemaphoreType.DMA,
            pltpu.SemaphoreType.DMA,
            pltpu.SemaphoreType.DMA,
        ],
    )
    def edge_kernel(tab, na2d, nb2d, pE, pA, pI, consts, zeros8, ids2d,
                    l0_o, cc_o, ss_o, Ne_o, M1_o, M2_o, Ve_o, accp,
                    naidx, nbidx, rowsA, rowsB, pEv, pAv, pIv,
                    l0v, ccv, ssv, Nev, M1v, M2v, Vev, updA, updB, cvec,
                    zbuf, idbuf, acc_sh, semIn, semA, semB, semOut, semU):
        cid = lax.axis_index("c")
        sid = lax.axis_index("s")
        wid = sid * NC + cid

        pltpu.sync_copy(zeros8, zbuf)
        pltpu.sync_copy(ids2d.at[pl.ds(sid * KR, KR)], idbuf)

        def zfire(j, carry):
            pltpu.async_copy(
                zbuf.at[pl.ds(0, SS)], acc_sh.at[idbuf.at[j]], semU)
            return carry
        lax.fori_loop(0, KR, zfire, 0)

        def zdrain(j, carry):
            pltpu.make_async_copy(
                zbuf.at[pl.ds(0, SS)], acc_sh.at[idbuf.at[0]], semU).wait()
            return carry
        lax.fori_loop(0, KR, zdrain, 0)
        pltpu.sync_copy(consts, cvec)

        iot = lax.iota(jnp.int32, L)
        zf = jnp.zeros((L,), jnp.float32)

        def zpad(g, carry):
            r = g * L + iot
            for col in range(3, W):
                cvec_i = jnp.full((L,), col, jnp.int32)
                plsc.store_scatter(updA, [r, cvec_i], zf)
                plsc.store_scatter(updB, [r, cvec_i], zf)
            return carry
        lax.fori_loop(0, 2 * G, zpad, 0)

        plsc.subcore_barrier()

        s_ax = cvec[pl.ds(0, L)]
        s_bend = cvec[pl.ds(16, L)]
        s_sw = cvec[pl.ds(32, L)]
        s_mw = cvec[pl.ds(48, L)]
        Fc = cvec[pl.ds(64, L)]
        Mc = cvec[pl.ds(80, L)]

        base = wid * EPW
        base100 = wid * (EPW // SS)

        def fire_inputs(k):
            par = lax.rem(k, 3)
            off = base + k * C
            off100 = base100 + k * K100
            pltpu.async_copy(na2d.at[pl.ds(off100, K100)],
                             naidx.at[pl.ds(par * K100, K100)], semIn)
            pltpu.async_copy(nb2d.at[pl.ds(off100, K100)],
                             nbidx.at[pl.ds(par * K100, K100)], semIn)
            pltpu.async_copy(pE.at[pl.ds(off, C)],
                             pEv.at[pl.ds(par * C, C)], semIn)
            pltpu.async_copy(pA.at[pl.ds(off, C)],
                             pAv.at[pl.ds(par * C, C)], semIn)
            pltpu.async_copy(pI.at[pl.ds(off, C)],
                             pIv.at[pl.ds(par * C, C)], semIn)

        def drain_inputs():
            pltpu.make_async_copy(
                na2d.at[pl.ds(0, K100)], naidx.at[pl.ds(0, K100)],
                semIn).wait()
            pltpu.make_async_copy(
                nb2d.at[pl.ds(0, K100)], nbidx.at[pl.ds(0, K100)],
                semIn).wait()
            pltpu.make_async_copy(
                pE.at[pl.ds(0, C)], pEv.at[pl.ds(0, C)], semIn).wait()
            pltpu.make_async_copy(
                pA.at[pl.ds(0, C)], pAv.at[pl.ds(0, C)], semIn).wait()
            pltpu.make_async_copy(
                pI.at[pl.ds(0, C)], pIv.at[pl.ds(0, C)], semIn).wait()

        def fire_gathers(k):
            par = lax.rem(k, 3)

            def fire(j, carry2):
                pltpu.async_copy(
                    tab.at[naidx.at[par * K100 + j]],
                    rowsA.at[pl.ds(par * C + j * SS, SS)], semA)
                pltpu.async_copy(
                    tab.at[nbidx.at[par * K100 + j]],
                    rowsB.at[pl.ds(par * C + j * SS, SS)], semB)
                return carry2
            lax.fori_loop(0, K100, fire, 0)

        def drain_gathers():
            def drain(j, carry2):
                pltpu.make_async_copy(
                    tab.at[naidx.at[0]], rowsA.at[pl.ds(0, SS)], semA).wait()
                pltpu.make_async_copy(
                    tab.at[nbidx.at[0]], rowsB.at[pl.ds(0, SS)], semB).wait()
                return carry2
            lax.fori_loop(0, K100, drain, 0)

        def drain_outs():
            for o in (l0_o, cc_o, ss_o, Ne_o, M1_o, M2_o, Ve_o):
                pltpu.make_async_copy(
                    l0v.at[pl.ds(0, C)], o.at[pl.ds(base, C)], semOut).wait()

        def drain_scatter():
            def udrain(j, carry2):
                pltpu.make_async_copy(
                    updA.at[pl.ds(0, SS)], acc_sh.at[naidx.at[0]],
                    semU).wait()
                pltpu.make_async_copy(
                    updB.at[pl.ds(0, SS)], acc_sh.at[nbidx.at[0]],
                    semU).wait()
                return carry2
            lax.fori_loop(0, K100, udrain, 0)

        fire_inputs(0)
        drain_inputs()
        fire_gathers(0)
        fire_inputs(1)
        drain_inputs()
        fire_gathers(1)

        def chunk(k, carry):
            p3 = lax.rem(k, 3)
            p2 = lax.rem(k, 2)
            off = base + k * C

            @pl.when(k > 0)
            def _():
                drain_outs()
                drain_scatter()

            @pl.when(k + 2 < NCH)
            def _():
                fire_inputs(k + 2)

            drain_gathers()

            def grp(g, carry2):
                r = p3 * C + g * L + iot
                r2 = p2 * C + g * L + iot

                def gA(col):
                    return plsc.load_gather(
                        rowsA, [r, jnp.full((L,), col, jnp.int32)])

                def gB(col):
                    return plsc.load_gather(
                        rowsB, [r, jnp.full((L,), col, jnp.int32)])

                d0 = gA(0); d1 = gA(1); ta = -gA(2)
                d3 = gB(0); d4 = gB(1); tb = -gB(2)
                cxA = gA(3); czA = gA(4)
                cxB = gB(3); czB = gB(4)
                dx = cxB - cxA
                dz = czB - czA
                q = dx * dx + dz * dz
                y = _rsqrt(q)
                y2 = y * y
                y3 = y2 * y
                sl3 = pl.ds(p3 * C + g * L, L)
                sl = pl.ds(p2 * C + g * L, L)
                eE = pEv[sl3]; eA = pAv[sl3]; eI = pIv[sl3]
                EAv = eE * eA
                EIv = eE * eI
                k_ax = EAv * s_ax * y
                k_bend = EIv * s_bend * y
                k_sw = EIv * s_sw * y2
                k_tr = EIv * s_ax * y3
                k_mw = EIv * s_mw * y2
                cc = dx * y
                ss = dz * y
                ua = cc * d0 + ss * d1
                wa = cc * d1 - ss * d0
                ub = cc * d3 + ss * d4
                wb = cc * d4 - ss * d3
                f0 = k_ax * (ua - ub)
                wab = wa - wb
                tsum = ta + tb
                f1 = 12.0 * k_tr * wab + 6.0 * k_sw * tsum
                mwab = 6.0 * k_mw * wab
                f2 = mwab + k_bend * (4.0 * ta + 2.0 * tb)
                f5 = mwab + k_bend * (2.0 * ta + 4.0 * tb)
                gx = cc * f0 - ss * f1
                gy = ss * f0 + cc * f1
                l0v[sl] = q * y
                ccv[sl] = cc
                ssv[sl] = ss
                Nev[sl] = -f0 * Fc
                M1v[sl] = f2 * Mc
                M2v[sl] = f5 * Mc
                Vev[sl] = -f1 * Fc

                def cS(i):
                    return jnp.full((L,), i, jnp.int32)
                plsc.store_scatter(updA, [r2, cS(0)], gx)
                plsc.store_scatter(updA, [r2, cS(1)], gy)
                plsc.store_scatter(updA, [r2, cS(2)], f2)
                plsc.store_scatter(updB, [r2, cS(0)], -gx)
                plsc.store_scatter(updB, [r2, cS(1)], -gy)
                plsc.store_scatter(updB, [r2, cS(2)], f5)
                return carry2

            lax.fori_loop(0, G, grp, 0)

            ds_o = pl.ds(off, C)
            ds_p = pl.ds(p2 * C, C)
            pltpu.async_copy(l0v.at[ds_p], l0_o.at[ds_o], semOut)
            pltpu.async_copy(ccv.at[ds_p], cc_o.at[ds_o], semOut)
            pltpu.async_copy(ssv.at[ds_p], ss_o.at[ds_o], semOut)
            pltpu.async_copy(Nev.at[ds_p], Ne_o.at[ds_o], semOut)
            pltpu.async_copy(M1v.at[ds_p], M1_o.at[ds_o], semOut)
            pltpu.async_copy(M2v.at[ds_p], M2_o.at[ds_o], semOut)
            pltpu.async_copy(Vev.at[ds_p], Ve_o.at[ds_o], semOut)

            def ufire(j, carry2):
                pltpu.async_copy(
                    updA.at[pl.ds(p2 * C + j * SS, SS)],
                    acc_sh.at[naidx.at[p3 * K100 + j]], semU, add=True)
                pltpu.async_copy(
                    updB.at[pl.ds(p2 * C + j * SS, SS)],
                    acc_sh.at[nbidx.at[p3 * K100 + j]], semU, add=True)
                return carry2
            lax.fori_loop(0, K100, ufire, 0)

            @pl.when(k + 2 < NCH)
            def _():
                drain_inputs()
                fire_gathers(k + 2)
            return carry

        lax.fori_loop(0, NCH, chunk, 0)
        drain_outs()
        drain_scatter()

        plsc.subcore_barrier()
        for r in range(RPT // ZR):
            def gfire(j, carry, r=r):
                pltpu.async_copy(
                    acc_sh.at[idbuf.at[r * ZK + j]],
                    zbuf.at[pl.ds(j * SS, SS)], semU)
                return carry
            lax.fori_loop(0, ZK, gfire, 0)

            def gdrain(j, carry):
                pltpu.make_async_copy(
                    acc_sh.at[idbuf.at[0]], zbuf.at[pl.ds(0, SS)],
                    semU).wait()
                return carry
            lax.fori_loop(0, ZK, gdrain, 0)
            pltpu.sync_copy(
                zbuf, accp.at[cid, pl.ds(sid * RPT + r * ZR, ZR)])

    return edge_kernel


def _make_node_kernel(NPAD, CH2):
    FPT4 = NPAD * 4 // NW
    CH8 = CH2 * 2
    NCH2 = FPT4 // CH2
    G2 = CH2 // L
    G8 = CH8 // L

    mesh = plsc.VectorSubcoreMesh(core_axis_name="c", subcore_axis_name="s",
                                  num_cores=NC, num_subcores=NS)

    @functools.partial(
        pl.kernel,
        out_type=[jax.ShapeDtypeStruct((NPAD * 8,), jnp.float32),
                  jax.ShapeDtypeStruct((NPAD * 4,), jnp.float32),
                  jax.ShapeDtypeStruct((NPAD * 4,), jnp.float32)],
        mesh=mesh,
        compiler_params=pltpu.CompilerParams(
            needs_layout_passes=False, use_tc_tiling_on_sc=False),
        scratch_types=[
            pltpu.VMEM((CH8,), jnp.float32),
            pltpu.VMEM((CH8,), jnp.float32),
            pltpu.VMEM((CH8,), jnp.float32),
            pltpu.VMEM((CH2,), jnp.float32),
            pltpu.VMEM((CH2,), jnp.float32),
            pltpu.VMEM((CH2,), jnp.float32),
            pltpu.VMEM((CH2,), jnp.float32),
            pltpu.VMEM((32,), jnp.float32),
        ],
    )
    def node_kernel(accpf, pr4f, fx4f, scl, nf_o, pd_o, fe_o,
                    a0v, a1v, nfv, prv, fxv, pdv, fev, sclv):
        cid = lax.axis_index("c")
        sid = lax.axis_index("s")
        wid = sid * NC + cid
        pltpu.sync_copy(scl, sclv)
        sdisp = sclv[pl.ds(0, L)]
        sfext = sclv[pl.ds(16, L)]
        base4 = wid * FPT4
        base8 = wid * FPT4 * 2

        def chunk(k, carry):
            off4 = base4 + k * CH2
            off8 = base8 + k * CH8
            pltpu.sync_copy(accpf.at[pl.ds(off8, CH8)], a0v)
            pltpu.sync_copy(accpf.at[pl.ds(NPAD * 8 + off8, CH8)], a1v)
            pltpu.sync_copy(pr4f.at[pl.ds(off4, CH2)], prv)
            pltpu.sync_copy(fx4f.at[pl.ds(off4, CH2)], fxv)

            def grp8(g, carry2):
                slg = pl.ds(g * L, L)
                nfv[slg] = a0v[slg] + a1v[slg]
                return carry2
            lax.fori_loop(0, G8, grp8, 0)

            def grp4(g, carry2):
                slg = pl.ds(g * L, L)
                pdv[slg] = prv[slg] * sdisp
                fev[slg] = fxv[slg] * sfext
                return carry2
            lax.fori_loop(0, G2, grp4, 0)

            pltpu.sync_copy(nfv, nf_o.at[pl.ds(off8, CH8)])
            pltpu.sync_copy(pdv, pd_o.at[pl.ds(off4, CH2)])
            pltpu.sync_copy(fev, fe_o.at[pl.ds(off4, CH2)])
            return carry

        lax.fori_loop(0, NCH2, chunk, 0)

    return node_kernel


def kernel(pred_raw, connectivity, coords, prop_E, prop_A, prop_I22,
           F_ext, u_c, theta_c, F_c, M_c):
    E = connectivity.shape[0]
    Nn = pred_raw.shape[0]
    C = 400
    NPAD = 102400

    nA = connectivity[:, 0].reshape(E // SS, SS)
    nB = connectivity[:, 1].reshape(E // SS, SS)
    tab = jnp.concatenate(
        [pred_raw, coords[:, 0:1], coords[:, 2:3],
         jnp.zeros((Nn, 3), jnp.float32)], axis=1)
    zeros8 = jnp.zeros((ZR, W), jnp.float32)

    def sp(v):
        return jnp.full((L,), v, jnp.float32)
    consts = jnp.concatenate([
        sp(u_c / F_c), sp(theta_c / M_c), sp(theta_c / F_c),
        sp(u_c / M_c), sp(F_c), sp(M_c)])

    z = jnp.float32(0.0)
    scl = jnp.concatenate([
        jnp.tile(jnp.stack([u_c, u_c, theta_c, z]), 4),
        jnp.tile(jnp.stack([1.0 / F_c, 1.0 / F_c, 1.0 / M_c, z]), 4)])

    pr4 = jnp.pad(pred_raw, ((0, NPAD - Nn), (0, 1)))
    fx4 = jnp.pad(F_ext, ((0, NPAD - Nn), (0, 1)))
    ids2d = jnp.arange(NPAD, dtype=jnp.int32).reshape(NPAD // SS, SS)

    edge_k = _make_edge_kernel(E, NPAD, C)
    l0, cc, ss, Ne, M1, M2, Ve, accp = edge_k(
        tab, nA, nB, prop_E, prop_A, prop_I22, consts, zeros8, ids2d)

    node_k = _make_node_kernel(NPAD, 3200)
    nf8f, pd4f, fe4f = node_k(
        accp.reshape(-1), pr4.reshape(-1), fx4.reshape(-1), scl)

    nf = nf8f.reshape(NPAD, W)[:Nn, :3]
    pd = pd4f.reshape(NPAD, 4)[:Nn, :3]
    fe = fe4f.reshape(NPAD, 4)[:Nn, :3]
    return (nf, fe, Ne, M1, M2, Ve, pd, l0, cc, ss)
